# Initial kernel scaffold; baseline (speedup 1.0000x reference)
import jax
import jax.numpy as jnp
from jax.experimental import pallas as pl


def kernel(x, edge_index, edge_attr, batch, global_features, params):
    raise NotImplementedError("write your pallas kernel here")


# speedup vs baseline: 2.2124x; 2.2124x over previous
import functools

import jax
import jax.numpy as jnp
from jax import lax
from jax.experimental import pallas as pl
from jax.experimental.pallas import tpu as pltpu
from jax.experimental.pallas import tpu_sc as plsc

_N = 10000
_E = 320000
_F = 128
_DE = 16
_H = 128
_NG = 16
_GF = 32

_NP = 10112
_NSUB = 16
_NCORE = 2
_HHALF = 64
_CH = 128
_T = 157
_EPS = _CH * _T
_EP = _EPS * _NSUB
_RPS = _NP // _NSUB
_ZR = _RPS // 4

_EBLK = 2048
_NBLK = 1264


def _gelu(v):
    return 0.5 * v * (1.0 + lax.erf(v * 0.7071067811865476))


def _edge_lin_body(ea_ref, wlo_ref, whi_ref, blo_ref, bhi_ref, out_ref):
    ea = ea_ref[...]
    out_ref[0] = (jnp.dot(ea, wlo_ref[...], preferred_element_type=jnp.float32)
                  + blo_ref[...])
    out_ref[1] = (jnp.dot(ea, whi_ref[...], preferred_element_type=jnp.float32)
                  + bhi_ref[...])


def _edge_lin(ea, wlo, whi, blo, bhi):
    zz = lambda i: (0, 0)
    return pl.pallas_call(
        _edge_lin_body,
        grid=(_EP // _EBLK,),
        in_specs=[
            pl.BlockSpec((_EBLK, _DE), lambda i: (i, 0)),
            pl.BlockSpec((_DE, _HHALF), zz),
            pl.BlockSpec((_DE, _HHALF), zz),
            pl.BlockSpec((1, _HHALF), zz),
            pl.BlockSpec((1, _HHALF), zz),
        ],
        out_specs=pl.BlockSpec((_NCORE, _EBLK, _HHALF), lambda i: (0, i, 0)),
        out_shape=jax.ShapeDtypeStruct((_NCORE, _EP, _HHALF), jnp.float32),
    )(ea, wlo, whi, blo, bhi)


@functools.partial(
    pl.kernel,
    out_type=jax.ShapeDtypeStruct((_NCORE, _NP, _HHALF), jnp.float32),
    mesh=plsc.VectorSubcoreMesh(core_axis_name="c", subcore_axis_name="s"),
    scratch_types=[
        pltpu.VMEM_SHARED((_NP, _HHALF), jnp.float32),
        pltpu.VMEM_SHARED((_NP, _HHALF), jnp.float32),
        pltpu.VMEM((_CH,), jnp.int32),
        pltpu.VMEM((_CH,), jnp.int32),
        pltpu.VMEM((_CH, _HHALF), jnp.float32),
        pltpu.VMEM((_CH, _HHALF), jnp.float32),
        pltpu.SemaphoreType.DMA,
    ],
)
def _sc_edge(h_hbm, e_hbm, src_hbm, dst_hbm, out_hbm,
             sh_h, sh_acc, src_v, dst_v, e_v, rows_v, sem):
    c = lax.axis_index("c")
    s = lax.axis_index("s")
    row0 = s * _RPS

    pltpu.sync_copy(
        h_hbm.at[c, pl.ds(row0, _RPS)],
        sh_h.at[pl.ds(row0, _RPS)],
    )

    def _zb(i, carry):
        for j in range(_HHALF // 16):
            e_v[i, pl.ds(j * 16, 16)] = jnp.zeros((16,), jnp.float32)
        return carry

    lax.fori_loop(0, _CH, _zb, 0)
    for k in range(_RPS // _CH):
        pltpu.sync_copy(e_v, sh_acc.at[pl.ds(row0 + k * _CH, _CH)])
    _ZREM = _RPS - (_RPS // _CH) * _CH
    if _ZREM:
        pltpu.sync_copy(
            e_v.at[pl.ds(0, _ZREM)],
            sh_acc.at[pl.ds(row0 + (_RPS // _CH) * _CH, _ZREM)],
        )
    plsc.subcore_barrier()

    e0 = s * _EPS

    def _chunk(t, carry):
        off = e0 + t * _CH
        pltpu.sync_copy(src_hbm.at[pl.ds(off, _CH)], src_v)
        pltpu.sync_copy(dst_hbm.at[pl.ds(off, _CH)], dst_v)
        pltpu.sync_copy(e_hbm.at[c, pl.ds(off, _CH)], e_v)
        pltpu.async_copy(sh_h.at[src_v], rows_v, sem).wait()

        def _cb(i, cc):
            for j in range(_HHALF // 16):
                sl = pl.ds(j * 16, 16)
                rows_v[i, sl] = jnp.maximum(rows_v[i, sl] + e_v[i, sl], 0.0)
            return cc

        lax.fori_loop(0, _CH, _cb, 0)
        pltpu.sync_copy(rows_v, sh_acc.at[dst_v], add=True)
        return carry

    lax.fori_loop(0, _T, _chunk, 0)

    plsc.subcore_barrier()
    pltpu.sync_copy(
        sh_acc.at[pl.ds(row0, _RPS)], out_hbm.at[c, pl.ds(row0, _RPS)]
    )


def _node_mlp_body(hlo_ref, hhi_ref, alo_ref, ahi_ref, sc_ref, w1_ref, b1_ref,
                   w2_ref, b2_ref, g_ref, b_ref, out_ref):
    h = jnp.concatenate([hlo_ref[0], hhi_ref[0]], axis=1)
    aggr = jnp.concatenate([alo_ref[0], ahi_ref[0]], axis=1)
    z = sc_ref[...] * h + aggr
    q = _gelu(jnp.dot(z, w1_ref[...], preferred_element_type=jnp.float32)
              + b1_ref[...])
    q = jnp.dot(q, w2_ref[...], preferred_element_type=jnp.float32) + b2_ref[...]
    mu = jnp.mean(q, axis=1, keepdims=True)
    var = jnp.mean((q - mu) ** 2, axis=1, keepdims=True)
    q = (q - mu) * lax.rsqrt(var + 1e-5) * g_ref[...] + b_ref[...]
    q = _gelu(q)
    out_ref[0] = q[:, :_HHALF]
    out_ref[1] = q[:, _HHALF:]


def _node_mlp(hs, aggr, scale, w1, b1, w2, b2, g, b):
    zz = lambda i: (0, 0)
    return pl.pallas_call(
        _node_mlp_body,
        grid=(_NP // _NBLK,),
        in_specs=[
            pl.BlockSpec((1, _NBLK, _HHALF), lambda i: (0, i, 0)),
            pl.BlockSpec((1, _NBLK, _HHALF), lambda i: (1, i, 0)),
            pl.BlockSpec((1, _NBLK, _HHALF), lambda i: (0, i, 0)),
            pl.BlockSpec((1, _NBLK, _HHALF), lambda i: (1, i, 0)),
            pl.BlockSpec((1, 1), zz),
            pl.BlockSpec((_F, _H), zz),
            pl.BlockSpec((1, _H), zz),
            pl.BlockSpec((_H, _H), zz),
            pl.BlockSpec((1, _H), zz),
            pl.BlockSpec((1, _H), zz),
            pl.BlockSpec((1, _H), zz),
        ],
        out_specs=pl.BlockSpec((_NCORE, _NBLK, _HHALF), lambda i: (0, i, 0)),
        out_shape=jax.ShapeDtypeStruct((_NCORE, _NP, _HHALF), jnp.float32),
    )(hs, hs, aggr, aggr, scale, w1, b1, w2, b2, g, b)


def _readout_body(h_ref, bcol_ref, brow_ref, gf_ref, wg1_ref, bg1_ref,
                  wg2_ref, bg2_ref, lg_ref, lb_ref, w1_ref, b1_ref,
                  w2_ref, b2_ref, w3_ref, b3_ref, out_ref):
    h = jnp.concatenate([h_ref[0], h_ref[1]], axis=1)
    g1 = _gelu(jnp.dot(h, wg1_ref[...], preferred_element_type=jnp.float32)
               + bg1_ref[...])
    gate = (jnp.dot(g1, wg2_ref[...], preferred_element_type=jnp.float32)
            + bg2_ref[...])
    onehot = (bcol_ref[...]
              == lax.broadcasted_iota(jnp.int32, (_NP, _NG), 1)
              ).astype(jnp.float32)
    m = jnp.max(jnp.where(onehot > 0, gate, jnp.float32(-1e30)), axis=0)
    m2 = m.reshape(_NG, 1)
    ex = jnp.exp(gate - jnp.dot(onehot, m2, preferred_element_type=jnp.float32))
    valid = (lax.broadcasted_iota(jnp.int32, (_NP, 1), 0) < _N
             ).astype(jnp.float32)
    ex = ex * valid
    den = jnp.sum(ex * onehot, axis=0).reshape(_NG, 1)
    den = jnp.where(den > 0, den, jnp.float32(1.0))
    alpha = ex * jnp.dot(onehot, 1.0 / den, preferred_element_type=jnp.float32)
    onehot_t = (brow_ref[...]
                == lax.broadcasted_iota(jnp.int32, (_NG, _NP), 0)
                ).astype(jnp.float32)
    pooled = jnp.dot(onehot_t, alpha * h, preferred_element_type=jnp.float32)
    cvec = jnp.concatenate([pooled, gf_ref[...]], axis=1)
    mu = jnp.mean(cvec, axis=1, keepdims=True)
    var = jnp.mean((cvec - mu) ** 2, axis=1, keepdims=True)
    cvec = (cvec - mu) * lax.rsqrt(var + 1e-5) * lg_ref[...] + lb_ref[...]
    cvec = _gelu(jnp.dot(cvec, w1_ref[...], preferred_element_type=jnp.float32)
                 + b1_ref[...])
    cvec = _gelu(jnp.dot(cvec, w2_ref[...], preferred_element_type=jnp.float32)
                 + b2_ref[...])
    out_ref[...] = (jnp.dot(cvec, w3_ref[...],
                            preferred_element_type=jnp.float32) + b3_ref[...])


def _readout(h, bcol, brow, gf, gp, cp):
    return pl.pallas_call(
        _readout_body,
        out_shape=jax.ShapeDtypeStruct((_NG, 2), jnp.float32),
    )(h, bcol, brow, gf,
      gp["W1"], gp["b1"].reshape(1, -1), gp["W2"], gp["b2"].reshape(1, -1),
      cp["ln_g"].reshape(1, -1), cp["ln_b"].reshape(1, -1),
      cp["W1"], cp["b1"].reshape(1, -1), cp["W2"], cp["b2"].reshape(1, -1),
      cp["W3"], cp["b3"].reshape(1, -1))


def kernel(x, edge_index, edge_attr, batch, global_features, params):
    f32 = jnp.float32
    src = edge_index[0]
    dst = edge_index[1]
    pad_e = _EP - _E
    srcp = jnp.concatenate([src, jnp.full((pad_e,), _N, jnp.int32)])
    dstp = jnp.concatenate([dst, jnp.full((pad_e,), _N, jnp.int32)])
    eap = jnp.concatenate([edge_attr, jnp.zeros((pad_e, _DE), f32)], axis=0)
    xpad = jnp.concatenate([x, jnp.zeros((_NP - _N, _F), f32)], axis=0)
    hs = jnp.stack([xpad[:, :_HHALF], xpad[:, _HHALF:]], axis=0)
    bpad = jnp.concatenate([batch, jnp.full((_NP - _N,), _NG, jnp.int32)])
    bcol = bpad.reshape(_NP, 1)
    brow = bpad.reshape(1, _NP)

    for p in params["layers"]:
        e = _edge_lin(eap, p["We"][:, :_HHALF], p["We"][:, _HHALF:],
                      p["be"][:_HHALF].reshape(1, _HHALF),
                      p["be"][_HHALF:].reshape(1, _HHALF))
        aggr = _sc_edge(hs, e, srcp, dstp)
        scale = jnp.reshape(1.0 + p["eps"], (1, 1)).astype(f32)
        hs = _node_mlp(hs, aggr, scale,
                       p["W1"], p["b1"].reshape(1, _H),
                       p["W2"], p["b2"].reshape(1, _H),
                       p["ln_g"].reshape(1, _H), p["ln_b"].reshape(1, _H))

    return _readout(hs, bcol, brow, global_features,
                    params["gate"], params["cls"])


# speedup vs baseline: 2.2143x; 1.0009x over previous
import functools

import jax
import jax.numpy as jnp
from jax import lax
from jax.experimental import pallas as pl
from jax.experimental.pallas import tpu as pltpu
from jax.experimental.pallas import tpu_sc as plsc

_N = 10000
_E = 320000
_F = 128
_DE = 16
_H = 128
_NG = 16
_GF = 32

_NP = 10112
_NSUB = 16
_NCORE = 2
_HHALF = 64
_CH = 128
_CPS = 157
_EPSUB = _CPS * _CH
_EP = _EPSUB * _NSUB
_RPS = _NP // _NSUB

_EBLK = 2048
_NBLK = 1264


def _gelu(v):
    return 0.5 * v * (1.0 + lax.erf(v * 0.7071067811865476))


def _edge_lin_body(ea_ref, wlo_ref, whi_ref, blo_ref, bhi_ref, out_ref):
    ea = ea_ref[...]
    out_ref[0] = (jnp.dot(ea, wlo_ref[...], preferred_element_type=jnp.float32)
                  + blo_ref[...])
    out_ref[1] = (jnp.dot(ea, whi_ref[...], preferred_element_type=jnp.float32)
                  + bhi_ref[...])


def _edge_lin(ea, wlo, whi, blo, bhi):
    zz = lambda i: (0, 0)
    return pl.pallas_call(
        _edge_lin_body,
        grid=(_EP // _EBLK,),
        in_specs=[
            pl.BlockSpec((_EBLK, _DE), lambda i: (i, 0)),
            pl.BlockSpec((_DE, _HHALF), zz),
            pl.BlockSpec((_DE, _HHALF), zz),
            pl.BlockSpec((1, _HHALF), zz),
            pl.BlockSpec((1, _HHALF), zz),
        ],
        out_specs=pl.BlockSpec((_NCORE, _EBLK, _HHALF), lambda i: (0, i, 0)),
        out_shape=jax.ShapeDtypeStruct((_NCORE, _EP, _HHALF), jnp.float32),
    )(ea, wlo, whi, blo, bhi)


@functools.partial(
    pl.kernel,
    out_type=jax.ShapeDtypeStruct((_NCORE, _NP, _HHALF), jnp.float32),
    mesh=plsc.VectorSubcoreMesh(core_axis_name="c", subcore_axis_name="s"),
    scratch_types=[
        pltpu.VMEM_SHARED((_NP, _HHALF), jnp.float32),
        pltpu.VMEM_SHARED((_NP, _HHALF), jnp.float32),
        pltpu.VMEM((1, _CH), jnp.int32),
        pltpu.VMEM((1, _CH), jnp.int32),
        pltpu.VMEM((_CH, _HHALF), jnp.float32),
        pltpu.VMEM((_CH, _HHALF), jnp.float32),
        pltpu.SemaphoreType.DMA,
        pltpu.SemaphoreType.DMA,
    ],
)
def _sc_edge(h_hbm, e_hbm, src_hbm, dst_hbm, out_hbm,
             sh_h, sh_acc, si, di, ev, rv, sg, ss):
    c = lax.axis_index("c")
    s = lax.axis_index("s")
    row0 = s * _RPS

    pltpu.sync_copy(
        h_hbm.at[c, pl.ds(row0, _RPS)],
        sh_h.at[pl.ds(row0, _RPS)],
    )

    def _zb(i, carry):
        for j in range(_HHALF // 16):
            ev[i, pl.ds(j * 16, 16)] = jnp.zeros((16,), jnp.float32)
        return carry

    lax.fori_loop(0, _CH, _zb, 0)
    for k in range(_RPS // _CH):
        pltpu.sync_copy(ev, sh_acc.at[pl.ds(row0 + k * _CH, _CH)])
    _ZREM = _RPS - (_RPS // _CH) * _CH
    if _ZREM:
        pltpu.sync_copy(
            ev.at[pl.ds(0, _ZREM)],
            sh_acc.at[pl.ds(row0 + (_RPS // _CH) * _CH, _ZREM)],
        )
    plsc.subcore_barrier()

    ebase = s * _EPSUB
    crow = s * _CPS

    def _chunk(kg, carry):
        pltpu.sync_copy(src_hbm.at[pl.ds(crow + kg, 1)], si)
        pltpu.sync_copy(dst_hbm.at[pl.ds(crow + kg, 1)], di)
        pltpu.sync_copy(e_hbm.at[c, pl.ds(ebase + kg * _CH, _CH)], ev)

        pltpu.async_copy(sh_h.at[si.at[0]], rv, sg)
        pltpu.make_async_copy(sh_h.at[si.at[0]], rv, sg).wait()

        def _cb(j, cc):
            for j4 in range(_HHALF // 16):
                sl = pl.ds(j4 * 16, 16)
                rv[j, sl] = jnp.maximum(rv[j, sl] + ev[j, sl], 0.0)
            return cc

        lax.fori_loop(0, _CH, _cb, 0)

        pltpu.async_copy(rv, sh_acc.at[di.at[0]], ss, add=True)
        pltpu.make_async_copy(rv, sh_acc.at[di.at[0]], ss).wait()
        return carry

    lax.fori_loop(0, _CPS, _chunk, 0)

    plsc.subcore_barrier()
    pltpu.sync_copy(
        sh_acc.at[pl.ds(row0, _RPS)], out_hbm.at[c, pl.ds(row0, _RPS)]
    )


def _node_mlp_body(hlo_ref, hhi_ref, alo_ref, ahi_ref, sc_ref, w1_ref, b1_ref,
                   w2_ref, b2_ref, g_ref, b_ref, out_ref):
    h = jnp.concatenate([hlo_ref[0], hhi_ref[0]], axis=1)
    aggr = jnp.concatenate([alo_ref[0], ahi_ref[0]], axis=1)
    z = sc_ref[...] * h + aggr
    q = _gelu(jnp.dot(z, w1_ref[...], preferred_element_type=jnp.float32)
              + b1_ref[...])
    q = jnp.dot(q, w2_ref[...], preferred_element_type=jnp.float32) + b2_ref[...]
    mu = jnp.mean(q, axis=1, keepdims=True)
    var = jnp.mean((q - mu) ** 2, axis=1, keepdims=True)
    q = (q - mu) * lax.rsqrt(var + 1e-5) * g_ref[...] + b_ref[...]
    q = _gelu(q)
    out_ref[0] = q[:, :_HHALF]
    out_ref[1] = q[:, _HHALF:]


def _node_mlp(hs, aggr, scale, w1, b1, w2, b2, g, b):
    zz = lambda i: (0, 0)
    return pl.pallas_call(
        _node_mlp_body,
        grid=(_NP // _NBLK,),
        in_specs=[
            pl.BlockSpec((1, _NBLK, _HHALF), lambda i: (0, i, 0)),
            pl.BlockSpec((1, _NBLK, _HHALF), lambda i: (1, i, 0)),
            pl.BlockSpec((1, _NBLK, _HHALF), lambda i: (0, i, 0)),
            pl.BlockSpec((1, _NBLK, _HHALF), lambda i: (1, i, 0)),
            pl.BlockSpec((1, 1), zz),
            pl.BlockSpec((_F, _H), zz),
            pl.BlockSpec((1, _H), zz),
            pl.BlockSpec((_H, _H), zz),
            pl.BlockSpec((1, _H), zz),
            pl.BlockSpec((1, _H), zz),
            pl.BlockSpec((1, _H), zz),
        ],
        out_specs=pl.BlockSpec((_NCORE, _NBLK, _HHALF), lambda i: (0, i, 0)),
        out_shape=jax.ShapeDtypeStruct((_NCORE, _NP, _HHALF), jnp.float32),
    )(hs, hs, aggr, aggr, scale, w1, b1, w2, b2, g, b)


def _readout_body(h_ref, bcol_ref, brow_ref, gf_ref, wg1_ref, bg1_ref,
                  wg2_ref, bg2_ref, lg_ref, lb_ref, w1_ref, b1_ref,
                  w2_ref, b2_ref, w3_ref, b3_ref, out_ref):
    h = jnp.concatenate([h_ref[0], h_ref[1]], axis=1)
    g1 = _gelu(jnp.dot(h, wg1_ref[...], preferred_element_type=jnp.float32)
               + bg1_ref[...])
    gate = (jnp.dot(g1, wg2_ref[...], preferred_element_type=jnp.float32)
            + bg2_ref[...])
    onehot = (bcol_ref[...]
              == lax.broadcasted_iota(jnp.int32, (_NP, _NG), 1)
              ).astype(jnp.float32)
    m = jnp.max(jnp.where(onehot > 0, gate, jnp.float32(-1e30)), axis=0)
    m2 = m.reshape(_NG, 1)
    ex = jnp.exp(gate - jnp.dot(onehot, m2, preferred_element_type=jnp.float32))
    valid = (lax.broadcasted_iota(jnp.int32, (_NP, 1), 0) < _N
             ).astype(jnp.float32)
    ex = ex * valid
    den = jnp.sum(ex * onehot, axis=0).reshape(_NG, 1)
    den = jnp.where(den > 0, den, jnp.float32(1.0))
    alpha = ex * jnp.dot(onehot, 1.0 / den, preferred_element_type=jnp.float32)
    onehot_t = (brow_ref[...]
                == lax.broadcasted_iota(jnp.int32, (_NG, _NP), 0)
                ).astype(jnp.float32)
    pooled = jnp.dot(onehot_t, alpha * h, preferred_element_type=jnp.float32)
    cvec = jnp.concatenate([pooled, gf_ref[...]], axis=1)
    mu = jnp.mean(cvec, axis=1, keepdims=True)
    var = jnp.mean((cvec - mu) ** 2, axis=1, keepdims=True)
    cvec = (cvec - mu) * lax.rsqrt(var + 1e-5) * lg_ref[...] + lb_ref[...]
    cvec = _gelu(jnp.dot(cvec, w1_ref[...], preferred_element_type=jnp.float32)
                 + b1_ref[...])
    cvec = _gelu(jnp.dot(cvec, w2_ref[...], preferred_element_type=jnp.float32)
                 + b2_ref[...])
    out_ref[...] = (jnp.dot(cvec, w3_ref[...],
                            preferred_element_type=jnp.float32) + b3_ref[...])


def _readout(h, bcol, brow, gf, gp, cp):
    return pl.pallas_call(
        _readout_body,
        out_shape=jax.ShapeDtypeStruct((_NG, 2), jnp.float32),
    )(h, bcol, brow, gf,
      gp["W1"], gp["b1"].reshape(1, -1), gp["W2"], gp["b2"].reshape(1, -1),
      cp["ln_g"].reshape(1, -1), cp["ln_b"].reshape(1, -1),
      cp["W1"], cp["b1"].reshape(1, -1), cp["W2"], cp["b2"].reshape(1, -1),
      cp["W3"], cp["b3"].reshape(1, -1))


def kernel(x, edge_index, edge_attr, batch, global_features, params):
    f32 = jnp.float32
    src = edge_index[0]
    dst = edge_index[1]
    pad_e = _EP - _E
    srcp = jnp.concatenate(
        [src, jnp.full((pad_e,), _N, jnp.int32)]).reshape(_EP // _CH, _CH)
    dstp = jnp.concatenate(
        [dst, jnp.full((pad_e,), _N, jnp.int32)]).reshape(_EP // _CH, _CH)
    eap = jnp.concatenate([edge_attr, jnp.zeros((pad_e, _DE), f32)], axis=0)
    xpad = jnp.concatenate([x, jnp.zeros((_NP - _N, _F), f32)], axis=0)
    hs = jnp.stack([xpad[:, :_HHALF], xpad[:, _HHALF:]], axis=0)
    bpad = jnp.concatenate([batch, jnp.full((_NP - _N,), _NG, jnp.int32)])
    bcol = bpad.reshape(_NP, 1)
    brow = bpad.reshape(1, _NP)

    for p in params["layers"]:
        e = _edge_lin(eap, p["We"][:, :_HHALF], p["We"][:, _HHALF:],
                      p["be"][:_HHALF].reshape(1, _HHALF),
                      p["be"][_HHALF:].reshape(1, _HHALF))
        aggr = _sc_edge(hs, e, srcp, dstp)
        scale = jnp.reshape(1.0 + p["eps"], (1, 1)).astype(f32)
        hs = _node_mlp(hs, aggr, scale,
                       p["W1"], p["b1"].reshape(1, _H),
                       p["W2"], p["b2"].reshape(1, _H),
                       p["ln_g"].reshape(1, _H), p["ln_b"].reshape(1, _H))

    return _readout(hs, bcol, brow, global_features,
                    params["gate"], params["cls"])


# speedup vs baseline: 2.8591x; 1.2912x over previous
import functools

import jax
import jax.numpy as jnp
from jax import lax
from jax.experimental import pallas as pl
from jax.experimental.pallas import tpu as pltpu
from jax.experimental.pallas import tpu_sc as plsc

_N = 10000
_E = 320000
_F = 128
_DE = 16
_H = 128
_NG = 16
_GF = 32

_NP = 10112
_NSUB = 16
_NCORE = 2
_HHALF = 64
_CH = 128
_CPS = 157
_EPSUB = _CPS * _CH
_EP = _EPSUB * _NSUB
_RPS = _NP // _NSUB

_EBLK = 2048
_NBLK = 1264


def _gelu(v):
    return 0.5 * v * (1.0 + lax.erf(v * 0.7071067811865476))


def _edge_lin_body(ea_ref, wlo_ref, whi_ref, blo_ref, bhi_ref, out_ref):
    ea = ea_ref[...]
    out_ref[0] = (jnp.dot(ea, wlo_ref[...], preferred_element_type=jnp.float32)
                  + blo_ref[...])
    out_ref[1] = (jnp.dot(ea, whi_ref[...], preferred_element_type=jnp.float32)
                  + bhi_ref[...])


def _edge_lin(ea, wlo, whi, blo, bhi):
    zz = lambda i: (0, 0)
    return pl.pallas_call(
        _edge_lin_body,
        grid=(_EP // _EBLK,),
        in_specs=[
            pl.BlockSpec((_EBLK, _DE), lambda i: (i, 0)),
            pl.BlockSpec((_DE, _HHALF), zz),
            pl.BlockSpec((_DE, _HHALF), zz),
            pl.BlockSpec((1, _HHALF), zz),
            pl.BlockSpec((1, _HHALF), zz),
        ],
        out_specs=pl.BlockSpec((_NCORE, _EBLK, _HHALF), lambda i: (0, i, 0)),
        out_shape=jax.ShapeDtypeStruct((_NCORE, _EP, _HHALF), jnp.float32),
    )(ea, wlo, whi, blo, bhi)


@functools.partial(
    pl.kernel,
    out_type=jax.ShapeDtypeStruct((_NCORE, _NP, _HHALF), jnp.float32),
    mesh=plsc.VectorSubcoreMesh(core_axis_name="c", subcore_axis_name="s"),
    scratch_types=[
        pltpu.VMEM_SHARED((_NP, _HHALF), jnp.float32),
        pltpu.VMEM_SHARED((_NP, _HHALF), jnp.float32),
        pltpu.VMEM((2, _CH), jnp.int32),
        pltpu.VMEM((_CH, _HHALF), jnp.float32),
        pltpu.VMEM((_CH, _HHALF), jnp.float32),
        pltpu.SemaphoreType.DMA,
        pltpu.SemaphoreType.DMA,
        pltpu.SemaphoreType.DMA,
    ],
)
def _sc_edge(h_hbm, e_hbm, idx_hbm, out_hbm,
             sh_h, sh_acc, si, ev, rv, se, sg, ss):
    c = lax.axis_index("c")
    s = lax.axis_index("s")
    row0 = s * _RPS

    pltpu.sync_copy(
        h_hbm.at[c, pl.ds(row0, _RPS)],
        sh_h.at[pl.ds(row0, _RPS)],
    )

    def _zb(i, carry):
        for j in range(_HHALF // 16):
            ev[i, pl.ds(j * 16, 16)] = jnp.zeros((16,), jnp.float32)
        return carry

    lax.fori_loop(0, _CH, _zb, 0)
    for k in range(_RPS // _CH):
        pltpu.sync_copy(ev, sh_acc.at[pl.ds(row0 + k * _CH, _CH)])
    _ZREM = _RPS - (_RPS // _CH) * _CH
    if _ZREM:
        pltpu.sync_copy(
            ev.at[pl.ds(0, _ZREM)],
            sh_acc.at[pl.ds(row0 + (_RPS // _CH) * _CH, _ZREM)],
        )
    plsc.subcore_barrier()

    ebase = s * _EPSUB
    crow = s * _CPS

    def _chunk(kg, carry):
        pltpu.async_copy(e_hbm.at[c, pl.ds(ebase + kg * _CH, _CH)], ev, se)
        pltpu.sync_copy(idx_hbm.at[crow + kg], si)

        pltpu.async_copy(sh_h.at[si.at[0]], rv, sg)
        pltpu.make_async_copy(
            e_hbm.at[c, pl.ds(ebase + kg * _CH, _CH)], ev, se).wait()
        pltpu.make_async_copy(sh_h.at[si.at[0]], rv, sg).wait()

        def _cb(j, cc):
            for j4 in range(_HHALF // 16):
                sl = pl.ds(j4 * 16, 16)
                rv[j, sl] = jnp.maximum(rv[j, sl] + ev[j, sl], 0.0)
            return cc

        lax.fori_loop(0, _CH, _cb, 0)

        pltpu.async_copy(rv, sh_acc.at[si.at[1]], ss, add=True)
        pltpu.make_async_copy(rv, sh_acc.at[si.at[1]], ss).wait()
        return carry

    lax.fori_loop(0, _CPS, _chunk, 0)

    plsc.subcore_barrier()
    pltpu.sync_copy(
        sh_acc.at[pl.ds(row0, _RPS)], out_hbm.at[c, pl.ds(row0, _RPS)]
    )


def _node_mlp_body(hlo_ref, hhi_ref, alo_ref, ahi_ref, sc_ref, w1_ref, b1_ref,
                   w2_ref, b2_ref, g_ref, b_ref, out_ref):
    h = jnp.concatenate([hlo_ref[0], hhi_ref[0]], axis=1)
    aggr = jnp.concatenate([alo_ref[0], ahi_ref[0]], axis=1)
    z = sc_ref[...] * h + aggr
    q = _gelu(jnp.dot(z, w1_ref[...], preferred_element_type=jnp.float32)
              + b1_ref[...])
    q = jnp.dot(q, w2_ref[...], preferred_element_type=jnp.float32) + b2_ref[...]
    mu = jnp.mean(q, axis=1, keepdims=True)
    var = jnp.mean((q - mu) ** 2, axis=1, keepdims=True)
    q = (q - mu) * lax.rsqrt(var + 1e-5) * g_ref[...] + b_ref[...]
    q = _gelu(q)
    out_ref[0] = q[:, :_HHALF]
    out_ref[1] = q[:, _HHALF:]


def _node_mlp(hs, aggr, scale, w1, b1, w2, b2, g, b):
    zz = lambda i: (0, 0)
    return pl.pallas_call(
        _node_mlp_body,
        grid=(_NP // _NBLK,),
        in_specs=[
            pl.BlockSpec((1, _NBLK, _HHALF), lambda i: (0, i, 0)),
            pl.BlockSpec((1, _NBLK, _HHALF), lambda i: (1, i, 0)),
            pl.BlockSpec((1, _NBLK, _HHALF), lambda i: (0, i, 0)),
            pl.BlockSpec((1, _NBLK, _HHALF), lambda i: (1, i, 0)),
            pl.BlockSpec((1, 1), zz),
            pl.BlockSpec((_F, _H), zz),
            pl.BlockSpec((1, _H), zz),
            pl.BlockSpec((_H, _H), zz),
            pl.BlockSpec((1, _H), zz),
            pl.BlockSpec((1, _H), zz),
            pl.BlockSpec((1, _H), zz),
        ],
        out_specs=pl.BlockSpec((_NCORE, _NBLK, _HHALF), lambda i: (0, i, 0)),
        out_shape=jax.ShapeDtypeStruct((_NCORE, _NP, _HHALF), jnp.float32),
    )(hs, hs, aggr, aggr, scale, w1, b1, w2, b2, g, b)


def _readout_body(h_ref, bcol_ref, brow_ref, gf_ref, wg1_ref, bg1_ref,
                  wg2_ref, bg2_ref, lg_ref, lb_ref, w1_ref, b1_ref,
                  w2_ref, b2_ref, w3_ref, b3_ref, out_ref):
    h = jnp.concatenate([h_ref[0], h_ref[1]], axis=1)
    g1 = _gelu(jnp.dot(h, wg1_ref[...], preferred_element_type=jnp.float32)
               + bg1_ref[...])
    gate = (jnp.dot(g1, wg2_ref[...], preferred_element_type=jnp.float32)
            + bg2_ref[...])
    onehot = (bcol_ref[...]
              == lax.broadcasted_iota(jnp.int32, (_NP, _NG), 1)
              ).astype(jnp.float32)
    m = jnp.max(jnp.where(onehot > 0, gate, jnp.float32(-1e30)), axis=0)
    m2 = m.reshape(_NG, 1)
    ex = jnp.exp(gate - jnp.dot(onehot, m2, preferred_element_type=jnp.float32))
    valid = (lax.broadcasted_iota(jnp.int32, (_NP, 1), 0) < _N
             ).astype(jnp.float32)
    ex = ex * valid
    den = jnp.sum(ex * onehot, axis=0).reshape(_NG, 1)
    den = jnp.where(den > 0, den, jnp.float32(1.0))
    alpha = ex * jnp.dot(onehot, 1.0 / den, preferred_element_type=jnp.float32)
    onehot_t = (brow_ref[...]
                == lax.broadcasted_iota(jnp.int32, (_NG, _NP), 0)
                ).astype(jnp.float32)
    pooled = jnp.dot(onehot_t, alpha * h, preferred_element_type=jnp.float32)
    cvec = jnp.concatenate([pooled, gf_ref[...]], axis=1)
    mu = jnp.mean(cvec, axis=1, keepdims=True)
    var = jnp.mean((cvec - mu) ** 2, axis=1, keepdims=True)
    cvec = (cvec - mu) * lax.rsqrt(var + 1e-5) * lg_ref[...] + lb_ref[...]
    cvec = _gelu(jnp.dot(cvec, w1_ref[...], preferred_element_type=jnp.float32)
                 + b1_ref[...])
    cvec = _gelu(jnp.dot(cvec, w2_ref[...], preferred_element_type=jnp.float32)
                 + b2_ref[...])
    out_ref[...] = (jnp.dot(cvec, w3_ref[...],
                            preferred_element_type=jnp.float32) + b3_ref[...])


def _readout(h, bcol, brow, gf, gp, cp):
    return pl.pallas_call(
        _readout_body,
        out_shape=jax.ShapeDtypeStruct((_NG, 2), jnp.float32),
    )(h, bcol, brow, gf,
      gp["W1"], gp["b1"].reshape(1, -1), gp["W2"], gp["b2"].reshape(1, -1),
      cp["ln_g"].reshape(1, -1), cp["ln_b"].reshape(1, -1),
      cp["W1"], cp["b1"].reshape(1, -1), cp["W2"], cp["b2"].reshape(1, -1),
      cp["W3"], cp["b3"].reshape(1, -1))


def kernel(x, edge_index, edge_attr, batch, global_features, params):
    f32 = jnp.float32
    src = edge_index[0]
    dst = edge_index[1]
    pad_e = _EP - _E
    srcp = jnp.concatenate(
        [src, jnp.full((pad_e,), _N, jnp.int32)]).reshape(_EP // _CH, _CH)
    dstp = jnp.concatenate(
        [dst, jnp.full((pad_e,), _N, jnp.int32)]).reshape(_EP // _CH, _CH)
    idxp = jnp.stack([srcp, dstp], axis=1)
    eap = jnp.concatenate([edge_attr, jnp.zeros((pad_e, _DE), f32)], axis=0)
    xpad = jnp.concatenate([x, jnp.zeros((_NP - _N, _F), f32)], axis=0)
    hs = jnp.stack([xpad[:, :_HHALF], xpad[:, _HHALF:]], axis=0)
    bpad = jnp.concatenate([batch, jnp.full((_NP - _N,), _NG, jnp.int32)])
    bcol = bpad.reshape(_NP, 1)
    brow = bpad.reshape(1, _NP)

    for p in params["layers"]:
        e = _edge_lin(eap, p["We"][:, :_HHALF], p["We"][:, _HHALF:],
                      p["be"][:_HHALF].reshape(1, _HHALF),
                      p["be"][_HHALF:].reshape(1, _HHALF))
        aggr = _sc_edge(hs, e, idxp)
        scale = jnp.reshape(1.0 + p["eps"], (1, 1)).astype(f32)
        hs = _node_mlp(hs, aggr, scale,
                       p["W1"], p["b1"].reshape(1, _H),
                       p["W2"], p["b2"].reshape(1, _H),
                       p["ln_g"].reshape(1, _H), p["ln_b"].reshape(1, _H))

    return _readout(hs, bcol, brow, global_features,
                    params["gate"], params["cls"])


# speedup vs baseline: 3.6309x; 1.2700x over previous
import functools

import jax
import jax.numpy as jnp
from jax import lax
from jax.experimental import pallas as pl
from jax.experimental.pallas import tpu as pltpu
from jax.experimental.pallas import tpu_sc as plsc

_N = 10000
_E = 320000
_F = 128
_DE = 16
_H = 128
_NG = 16
_GF = 32

_NP = 10112
_NSUB = 16
_NCORE = 2
_HHALF = 64
_CH = 128
_CPS = 158
_EPSUB = _CPS * _CH
_EP = _EPSUB * _NSUB
_RPS = _NP // _NSUB

_EBLK = 2048
_NBLK = 1264


def _gelu(v):
    return 0.5 * v * (1.0 + lax.erf(v * 0.7071067811865476))


def _edge_lin_body(ea_ref, wlo_ref, whi_ref, blo_ref, bhi_ref, out_ref):
    ea = ea_ref[...]
    out_ref[0] = (jnp.dot(ea, wlo_ref[...], preferred_element_type=jnp.float32)
                  + blo_ref[...])
    out_ref[1] = (jnp.dot(ea, whi_ref[...], preferred_element_type=jnp.float32)
                  + bhi_ref[...])


def _edge_lin(ea, wlo, whi, blo, bhi):
    zz = lambda i: (0, 0)
    return pl.pallas_call(
        _edge_lin_body,
        grid=(_EP // _EBLK,),
        in_specs=[
            pl.BlockSpec((_EBLK, _DE), lambda i: (i, 0)),
            pl.BlockSpec((_DE, _HHALF), zz),
            pl.BlockSpec((_DE, _HHALF), zz),
            pl.BlockSpec((1, _HHALF), zz),
            pl.BlockSpec((1, _HHALF), zz),
        ],
        out_specs=pl.BlockSpec((_NCORE, _EBLK, _HHALF), lambda i: (0, i, 0)),
        out_shape=jax.ShapeDtypeStruct((_NCORE, _EP, _HHALF), jnp.float32),
    )(ea, wlo, whi, blo, bhi)


@functools.partial(
    pl.kernel,
    out_type=jax.ShapeDtypeStruct((_NCORE, _NP, _HHALF), jnp.float32),
    mesh=plsc.VectorSubcoreMesh(core_axis_name="c", subcore_axis_name="s"),
    scratch_types=[
        pltpu.VMEM_SHARED((_NP, _HHALF), jnp.float32),
        pltpu.VMEM_SHARED((_NP, _HHALF), jnp.float32),
        pltpu.VMEM((2, _CH), jnp.int32),
        pltpu.VMEM((2, _CH), jnp.int32),
        pltpu.VMEM((_CH, _HHALF), jnp.float32),
        pltpu.VMEM((_CH, _HHALF), jnp.float32),
        pltpu.VMEM((_CH, _HHALF), jnp.float32),
        pltpu.SemaphoreType.DMA,
        pltpu.SemaphoreType.DMA,
        pltpu.SemaphoreType.DMA,
        pltpu.SemaphoreType.DMA,
        pltpu.SemaphoreType.DMA,
        pltpu.SemaphoreType.DMA,
    ],
)
def _sc_edge(h_hbm, e_hbm, idx_hbm, out_hbm,
             sh_h, sh_acc, si0, si1, ev0, ev1, rv,
             sx0, sx1, se0, se1, sg, ss):
    c = lax.axis_index("c")
    s = lax.axis_index("s")
    row0 = s * _RPS
    si = (si0, si1)
    ev = (ev0, ev1)
    sx = (sx0, sx1)
    se = (se0, se1)

    pltpu.sync_copy(
        h_hbm.at[c, pl.ds(row0, _RPS)],
        sh_h.at[pl.ds(row0, _RPS)],
    )

    def _zb(i, carry):
        for j in range(_HHALF // 16):
            ev0[i, pl.ds(j * 16, 16)] = jnp.zeros((16,), jnp.float32)
        return carry

    lax.fori_loop(0, _CH, _zb, 0)
    for k in range(_RPS // _CH):
        pltpu.sync_copy(ev0, sh_acc.at[pl.ds(row0 + k * _CH, _CH)])
    _ZREM = _RPS - (_RPS // _CH) * _CH
    if _ZREM:
        pltpu.sync_copy(
            ev0.at[pl.ds(0, _ZREM)],
            sh_acc.at[pl.ds(row0 + (_RPS // _CH) * _CH, _ZREM)],
        )
    plsc.subcore_barrier()

    ebase = s * _EPSUB
    crow = s * _CPS

    def _issue_idx_e(kg, p):
        pltpu.async_copy(idx_hbm.at[crow + kg], si[p], sx[p])
        pltpu.async_copy(e_hbm.at[c, pl.ds(ebase + kg * _CH, _CH)],
                         ev[p], se[p])

    def _wait_idx_e(kg, p):
        pltpu.make_async_copy(idx_hbm.at[crow + kg], si[p], sx[p]).wait()
        pltpu.make_async_copy(e_hbm.at[c, pl.ds(ebase + kg * _CH, _CH)],
                              ev[p], se[p]).wait()

    _issue_idx_e(0, 0)

    def _pair(i, carry):
        for p in range(2):
            kg = 2 * i + p
            q = 1 - p

            pltpu.make_async_copy(idx_hbm.at[crow + kg], si[p],
                                  sx[p]).wait()

            pltpu.async_copy(sh_h.at[si[p].at[0]], rv, sg)

            @pl.when(kg + 1 < _CPS)
            def _():
                _issue_idx_e(kg + 1, q)

            pltpu.make_async_copy(
                e_hbm.at[c, pl.ds(ebase + kg * _CH, _CH)], ev[p],
                se[p]).wait()
            pltpu.make_async_copy(sh_h.at[si[p].at[0]], rv, sg).wait()

            def _cb(j, cc):
                for j4 in range(_HHALF // 16):
                    sl = pl.ds(j4 * 16, 16)
                    rv[j, sl] = jnp.maximum(rv[j, sl] + ev[p][j, sl], 0.0)
                return cc

            lax.fori_loop(0, _CH, _cb, 0)

            pltpu.async_copy(rv, sh_acc.at[si[p].at[1]], ss, add=True)
            pltpu.make_async_copy(rv, sh_acc.at[si[p].at[1]], ss).wait()
        return carry

    lax.fori_loop(0, _CPS // 2, _pair, 0)

    plsc.subcore_barrier()
    pltpu.sync_copy(
        sh_acc.at[pl.ds(row0, _RPS)], out_hbm.at[c, pl.ds(row0, _RPS)]
    )


def _node_mlp_body(hlo_ref, hhi_ref, alo_ref, ahi_ref, sc_ref, w1_ref, b1_ref,
                   w2_ref, b2_ref, g_ref, b_ref, out_ref):
    h = jnp.concatenate([hlo_ref[0], hhi_ref[0]], axis=1)
    aggr = jnp.concatenate([alo_ref[0], ahi_ref[0]], axis=1)
    z = sc_ref[...] * h + aggr
    q = _gelu(jnp.dot(z, w1_ref[...], preferred_element_type=jnp.float32)
              + b1_ref[...])
    q = jnp.dot(q, w2_ref[...], preferred_element_type=jnp.float32) + b2_ref[...]
    mu = jnp.mean(q, axis=1, keepdims=True)
    var = jnp.mean((q - mu) ** 2, axis=1, keepdims=True)
    q = (q - mu) * lax.rsqrt(var + 1e-5) * g_ref[...] + b_ref[...]
    q = _gelu(q)
    out_ref[0] = q[:, :_HHALF]
    out_ref[1] = q[:, _HHALF:]


def _node_mlp(hs, aggr, scale, w1, b1, w2, b2, g, b):
    zz = lambda i: (0, 0)
    return pl.pallas_call(
        _node_mlp_body,
        grid=(_NP // _NBLK,),
        in_specs=[
            pl.BlockSpec((1, _NBLK, _HHALF), lambda i: (0, i, 0)),
            pl.BlockSpec((1, _NBLK, _HHALF), lambda i: (1, i, 0)),
            pl.BlockSpec((1, _NBLK, _HHALF), lambda i: (0, i, 0)),
            pl.BlockSpec((1, _NBLK, _HHALF), lambda i: (1, i, 0)),
            pl.BlockSpec((1, 1), zz),
            pl.BlockSpec((_F, _H), zz),
            pl.BlockSpec((1, _H), zz),
            pl.BlockSpec((_H, _H), zz),
            pl.BlockSpec((1, _H), zz),
            pl.BlockSpec((1, _H), zz),
            pl.BlockSpec((1, _H), zz),
        ],
        out_specs=pl.BlockSpec((_NCORE, _NBLK, _HHALF), lambda i: (0, i, 0)),
        out_shape=jax.ShapeDtypeStruct((_NCORE, _NP, _HHALF), jnp.float32),
    )(hs, hs, aggr, aggr, scale, w1, b1, w2, b2, g, b)


def _readout_body(h_ref, bcol_ref, brow_ref, gf_ref, wg1_ref, bg1_ref,
                  wg2_ref, bg2_ref, lg_ref, lb_ref, w1_ref, b1_ref,
                  w2_ref, b2_ref, w3_ref, b3_ref, out_ref):
    h = jnp.concatenate([h_ref[0], h_ref[1]], axis=1)
    g1 = _gelu(jnp.dot(h, wg1_ref[...], preferred_element_type=jnp.float32)
               + bg1_ref[...])
    gate = (jnp.dot(g1, wg2_ref[...], preferred_element_type=jnp.float32)
            + bg2_ref[...])
    onehot = (bcol_ref[...]
              == lax.broadcasted_iota(jnp.int32, (_NP, _NG), 1)
              ).astype(jnp.float32)
    m = jnp.max(jnp.where(onehot > 0, gate, jnp.float32(-1e30)), axis=0)
    m2 = m.reshape(_NG, 1)
    ex = jnp.exp(gate - jnp.dot(onehot, m2, preferred_element_type=jnp.float32))
    valid = (lax.broadcasted_iota(jnp.int32, (_NP, 1), 0) < _N
             ).astype(jnp.float32)
    ex = ex * valid
    den = jnp.sum(ex * onehot, axis=0).reshape(_NG, 1)
    den = jnp.where(den > 0, den, jnp.float32(1.0))
    alpha = ex * jnp.dot(onehot, 1.0 / den, preferred_element_type=jnp.float32)
    onehot_t = (brow_ref[...]
                == lax.broadcasted_iota(jnp.int32, (_NG, _NP), 0)
                ).astype(jnp.float32)
    pooled = jnp.dot(onehot_t, alpha * h, preferred_element_type=jnp.float32)
    cvec = jnp.concatenate([pooled, gf_ref[...]], axis=1)
    mu = jnp.mean(cvec, axis=1, keepdims=True)
    var = jnp.mean((cvec - mu) ** 2, axis=1, keepdims=True)
    cvec = (cvec - mu) * lax.rsqrt(var + 1e-5) * lg_ref[...] + lb_ref[...]
    cvec = _gelu(jnp.dot(cvec, w1_ref[...], preferred_element_type=jnp.float32)
                 + b1_ref[...])
    cvec = _gelu(jnp.dot(cvec, w2_ref[...], preferred_element_type=jnp.float32)
                 + b2_ref[...])
    out_ref[...] = (jnp.dot(cvec, w3_ref[...],
                            preferred_element_type=jnp.float32) + b3_ref[...])


def _readout(h, bcol, brow, gf, gp, cp):
    return pl.pallas_call(
        _readout_body,
        out_shape=jax.ShapeDtypeStruct((_NG, 2), jnp.float32),
    )(h, bcol, brow, gf,
      gp["W1"], gp["b1"].reshape(1, -1), gp["W2"], gp["b2"].reshape(1, -1),
      cp["ln_g"].reshape(1, -1), cp["ln_b"].reshape(1, -1),
      cp["W1"], cp["b1"].reshape(1, -1), cp["W2"], cp["b2"].reshape(1, -1),
      cp["W3"], cp["b3"].reshape(1, -1))


def kernel(x, edge_index, edge_attr, batch, global_features, params):
    f32 = jnp.float32
    src = edge_index[0]
    dst = edge_index[1]
    pad_e = _EP - _E
    srcp = jnp.concatenate(
        [src, jnp.full((pad_e,), _N, jnp.int32)]).reshape(_EP // _CH, _CH)
    dstp = jnp.concatenate(
        [dst, jnp.full((pad_e,), _N, jnp.int32)]).reshape(_EP // _CH, _CH)
    idxp = jnp.stack([srcp, dstp], axis=1)
    eap = jnp.concatenate([edge_attr, jnp.zeros((pad_e, _DE), f32)], axis=0)
    xpad = jnp.concatenate([x, jnp.zeros((_NP - _N, _F), f32)], axis=0)
    hs = jnp.stack([xpad[:, :_HHALF], xpad[:, _HHALF:]], axis=0)
    bpad = jnp.concatenate([batch, jnp.full((_NP - _N,), _NG, jnp.int32)])
    bcol = bpad.reshape(_NP, 1)
    brow = bpad.reshape(1, _NP)

    for p in params["layers"]:
        e = _edge_lin(eap, p["We"][:, :_HHALF], p["We"][:, _HHALF:],
                      p["be"][:_HHALF].reshape(1, _HHALF),
                      p["be"][_HHALF:].reshape(1, _HHALF))
        aggr = _sc_edge(hs, e, idxp)
        scale = jnp.reshape(1.0 + p["eps"], (1, 1)).astype(f32)
        hs = _node_mlp(hs, aggr, scale,
                       p["W1"], p["b1"].reshape(1, _H),
                       p["W2"], p["b2"].reshape(1, _H),
                       p["ln_g"].reshape(1, _H), p["ln_b"].reshape(1, _H))

    return _readout(hs, bcol, brow, global_features,
                    params["gate"], params["cls"])


# speedup vs baseline: 3.7388x; 1.0297x over previous
import functools

import jax
import jax.numpy as jnp
from jax import lax
from jax.experimental import pallas as pl
from jax.experimental.pallas import tpu as pltpu
from jax.experimental.pallas import tpu_sc as plsc

_N = 10000
_E = 320000
_F = 128
_DE = 16
_H = 128
_NG = 16
_GF = 32

_NP = 10112
_NSUB = 16
_NCORE = 2
_HHALF = 64
_CH = 128
_CPS = 158
_EPSUB = _CPS * _CH
_EP = _EPSUB * _NSUB
_RPS = _NP // _NSUB

_EBLK = 2048
_NBLK = 1264


def _gelu(v):
    return 0.5 * v * (1.0 + lax.erf(v * 0.7071067811865476))


def _edge_lin_body(ea_ref, wlo_ref, whi_ref, blo_ref, bhi_ref, out_ref):
    ea = ea_ref[...]
    out_ref[0] = (jnp.dot(ea, wlo_ref[...], preferred_element_type=jnp.float32)
                  + blo_ref[...])
    out_ref[1] = (jnp.dot(ea, whi_ref[...], preferred_element_type=jnp.float32)
                  + bhi_ref[...])


def _edge_lin(ea, wlo, whi, blo, bhi):
    zz = lambda i: (0, 0)
    return pl.pallas_call(
        _edge_lin_body,
        grid=(_EP // _EBLK,),
        in_specs=[
            pl.BlockSpec((_EBLK, _DE), lambda i: (i, 0)),
            pl.BlockSpec((_DE, _HHALF), zz),
            pl.BlockSpec((_DE, _HHALF), zz),
            pl.BlockSpec((1, _HHALF), zz),
            pl.BlockSpec((1, _HHALF), zz),
        ],
        out_specs=pl.BlockSpec((_NCORE, _EBLK, _HHALF), lambda i: (0, i, 0)),
        out_shape=jax.ShapeDtypeStruct((_NCORE, _EP, _HHALF), jnp.float32),
    )(ea, wlo, whi, blo, bhi)


@functools.partial(
    pl.kernel,
    out_type=jax.ShapeDtypeStruct((_NCORE, _NP, _HHALF), jnp.float32),
    mesh=plsc.VectorSubcoreMesh(core_axis_name="c", subcore_axis_name="s"),
    scratch_types=[
        pltpu.VMEM_SHARED((_NP, _HHALF), jnp.float32),
        pltpu.VMEM_SHARED((_NP, _HHALF), jnp.float32),
        pltpu.VMEM((2, _CH), jnp.int32),
        pltpu.VMEM((2, _CH), jnp.int32),
        pltpu.VMEM((_CH, _HHALF), jnp.float32),
        pltpu.VMEM((_CH, _HHALF), jnp.float32),
        pltpu.VMEM((_CH, _HHALF), jnp.float32),
        pltpu.SemaphoreType.DMA,
        pltpu.SemaphoreType.DMA,
        pltpu.SemaphoreType.DMA,
        pltpu.SemaphoreType.DMA,
        pltpu.SemaphoreType.DMA,
        pltpu.SemaphoreType.DMA,
        pltpu.SemaphoreType.DMA,
    ],
)
def _sc_edge(h_hbm, e_hbm, idx_hbm, out_hbm,
             sh_h, sh_acc, si0, si1, ev0, ev1, rv,
             sx0, sx1, se0, se1, sg, ss0, ss1):
    c = lax.axis_index("c")
    s = lax.axis_index("s")
    row0 = s * _RPS
    si = (si0, si1)
    ev = (ev0, ev1)
    sx = (sx0, sx1)
    se = (se0, se1)
    ss = (ss0, ss1)

    pltpu.sync_copy(
        h_hbm.at[c, pl.ds(row0, _RPS)],
        sh_h.at[pl.ds(row0, _RPS)],
    )

    def _zb(i, carry):
        for j in range(_HHALF // 16):
            ev0[i, pl.ds(j * 16, 16)] = jnp.zeros((16,), jnp.float32)
        return carry

    lax.fori_loop(0, _CH, _zb, 0)
    for k in range(_RPS // _CH):
        pltpu.sync_copy(ev0, sh_acc.at[pl.ds(row0 + k * _CH, _CH)])
    _ZREM = _RPS - (_RPS // _CH) * _CH
    if _ZREM:
        pltpu.sync_copy(
            ev0.at[pl.ds(0, _ZREM)],
            sh_acc.at[pl.ds(row0 + (_RPS // _CH) * _CH, _ZREM)],
        )
    plsc.subcore_barrier()

    ebase = s * _EPSUB
    crow = s * _CPS

    def _issue_idx_e(kg, p):
        pltpu.async_copy(idx_hbm.at[crow + kg], si[p], sx[p])
        pltpu.async_copy(e_hbm.at[c, pl.ds(ebase + kg * _CH, _CH)],
                         ev[p], se[p])

    def _wait_idx_e(kg, p):
        pltpu.make_async_copy(idx_hbm.at[crow + kg], si[p], sx[p]).wait()
        pltpu.make_async_copy(e_hbm.at[c, pl.ds(ebase + kg * _CH, _CH)],
                              ev[p], se[p]).wait()

    _issue_idx_e(0, 0)

    def _pair(i, carry):
        for p in range(2):
            kg = 2 * i + p
            q = 1 - p

            pltpu.make_async_copy(idx_hbm.at[crow + kg], si[p],
                                  sx[p]).wait()

            pltpu.async_copy(sh_h.at[si[p].at[0]], rv, sg)

            @pl.when(kg >= 1)
            def _():
                pltpu.make_async_copy(ev[q], sh_acc.at[si[q].at[1]],
                                      ss[q]).wait()

            @pl.when(kg + 1 < _CPS)
            def _():
                _issue_idx_e(kg + 1, q)

            pltpu.make_async_copy(
                e_hbm.at[c, pl.ds(ebase + kg * _CH, _CH)], ev[p],
                se[p]).wait()
            pltpu.make_async_copy(sh_h.at[si[p].at[0]], rv, sg).wait()

            def _cb(j, cc):
                for j4 in range(_HHALF // 16):
                    sl = pl.ds(j4 * 16, 16)
                    ev[p][j, sl] = jnp.maximum(
                        rv[j, sl] + ev[p][j, sl], 0.0)
                return cc

            lax.fori_loop(0, _CH, _cb, 0)

            pltpu.async_copy(ev[p], sh_acc.at[si[p].at[1]], ss[p],
                             add=True)
        return carry

    lax.fori_loop(0, _CPS // 2, _pair, 0)

    pltpu.make_async_copy(ev[1], sh_acc.at[si[1].at[1]], ss[1]).wait()

    plsc.subcore_barrier()
    pltpu.sync_copy(
        sh_acc.at[pl.ds(row0, _RPS)], out_hbm.at[c, pl.ds(row0, _RPS)]
    )


def _node_mlp_body(hlo_ref, hhi_ref, alo_ref, ahi_ref, sc_ref, w1_ref, b1_ref,
                   w2_ref, b2_ref, g_ref, b_ref, out_ref):
    h = jnp.concatenate([hlo_ref[0], hhi_ref[0]], axis=1)
    aggr = jnp.concatenate([alo_ref[0], ahi_ref[0]], axis=1)
    z = sc_ref[...] * h + aggr
    q = _gelu(jnp.dot(z, w1_ref[...], preferred_element_type=jnp.float32)
              + b1_ref[...])
    q = jnp.dot(q, w2_ref[...], preferred_element_type=jnp.float32) + b2_ref[...]
    mu = jnp.mean(q, axis=1, keepdims=True)
    var = jnp.mean((q - mu) ** 2, axis=1, keepdims=True)
    q = (q - mu) * lax.rsqrt(var + 1e-5) * g_ref[...] + b_ref[...]
    q = _gelu(q)
    out_ref[0] = q[:, :_HHALF]
    out_ref[1] = q[:, _HHALF:]


def _node_mlp(hs, aggr, scale, w1, b1, w2, b2, g, b):
    zz = lambda i: (0, 0)
    return pl.pallas_call(
        _node_mlp_body,
        grid=(_NP // _NBLK,),
        in_specs=[
            pl.BlockSpec((1, _NBLK, _HHALF), lambda i: (0, i, 0)),
            pl.BlockSpec((1, _NBLK, _HHALF), lambda i: (1, i, 0)),
            pl.BlockSpec((1, _NBLK, _HHALF), lambda i: (0, i, 0)),
            pl.BlockSpec((1, _NBLK, _HHALF), lambda i: (1, i, 0)),
            pl.BlockSpec((1, 1), zz),
            pl.BlockSpec((_F, _H), zz),
            pl.BlockSpec((1, _H), zz),
            pl.BlockSpec((_H, _H), zz),
            pl.BlockSpec((1, _H), zz),
            pl.BlockSpec((1, _H), zz),
            pl.BlockSpec((1, _H), zz),
        ],
        out_specs=pl.BlockSpec((_NCORE, _NBLK, _HHALF), lambda i: (0, i, 0)),
        out_shape=jax.ShapeDtypeStruct((_NCORE, _NP, _HHALF), jnp.float32),
    )(hs, hs, aggr, aggr, scale, w1, b1, w2, b2, g, b)


def _readout_body(h_ref, bcol_ref, brow_ref, gf_ref, wg1_ref, bg1_ref,
                  wg2_ref, bg2_ref, lg_ref, lb_ref, w1_ref, b1_ref,
                  w2_ref, b2_ref, w3_ref, b3_ref, out_ref):
    h = jnp.concatenate([h_ref[0], h_ref[1]], axis=1)
    g1 = _gelu(jnp.dot(h, wg1_ref[...], preferred_element_type=jnp.float32)
               + bg1_ref[...])
    gate = (jnp.dot(g1, wg2_ref[...], preferred_element_type=jnp.float32)
            + bg2_ref[...])
    onehot = (bcol_ref[...]
              == lax.broadcasted_iota(jnp.int32, (_NP, _NG), 1)
              ).astype(jnp.float32)
    m = jnp.max(jnp.where(onehot > 0, gate, jnp.float32(-1e30)), axis=0)
    m2 = m.reshape(_NG, 1)
    ex = jnp.exp(gate - jnp.dot(onehot, m2, preferred_element_type=jnp.float32))
    valid = (lax.broadcasted_iota(jnp.int32, (_NP, 1), 0) < _N
             ).astype(jnp.float32)
    ex = ex * valid
    den = jnp.sum(ex * onehot, axis=0).reshape(_NG, 1)
    den = jnp.where(den > 0, den, jnp.float32(1.0))
    alpha = ex * jnp.dot(onehot, 1.0 / den, preferred_element_type=jnp.float32)
    onehot_t = (brow_ref[...]
                == lax.broadcasted_iota(jnp.int32, (_NG, _NP), 0)
                ).astype(jnp.float32)
    pooled = jnp.dot(onehot_t, alpha * h, preferred_element_type=jnp.float32)
    cvec = jnp.concatenate([pooled, gf_ref[...]], axis=1)
    mu = jnp.mean(cvec, axis=1, keepdims=True)
    var = jnp.mean((cvec - mu) ** 2, axis=1, keepdims=True)
    cvec = (cvec - mu) * lax.rsqrt(var + 1e-5) * lg_ref[...] + lb_ref[...]
    cvec = _gelu(jnp.dot(cvec, w1_ref[...], preferred_element_type=jnp.float32)
                 + b1_ref[...])
    cvec = _gelu(jnp.dot(cvec, w2_ref[...], preferred_element_type=jnp.float32)
                 + b2_ref[...])
    out_ref[...] = (jnp.dot(cvec, w3_ref[...],
                            preferred_element_type=jnp.float32) + b3_ref[...])


def _readout(h, bcol, brow, gf, gp, cp):
    return pl.pallas_call(
        _readout_body,
        out_shape=jax.ShapeDtypeStruct((_NG, 2), jnp.float32),
    )(h, bcol, brow, gf,
      gp["W1"], gp["b1"].reshape(1, -1), gp["W2"], gp["b2"].reshape(1, -1),
      cp["ln_g"].reshape(1, -1), cp["ln_b"].reshape(1, -1),
      cp["W1"], cp["b1"].reshape(1, -1), cp["W2"], cp["b2"].reshape(1, -1),
      cp["W3"], cp["b3"].reshape(1, -1))


def kernel(x, edge_index, edge_attr, batch, global_features, params):
    f32 = jnp.float32
    src = edge_index[0]
    dst = edge_index[1]
    pad_e = _EP - _E
    srcp = jnp.concatenate(
        [src, jnp.full((pad_e,), _N, jnp.int32)]).reshape(_EP // _CH, _CH)
    dstp = jnp.concatenate(
        [dst, jnp.full((pad_e,), _N, jnp.int32)]).reshape(_EP // _CH, _CH)
    idxp = jnp.stack([srcp, dstp], axis=1)
    eap = jnp.concatenate([edge_attr, jnp.zeros((pad_e, _DE), f32)], axis=0)
    xpad = jnp.concatenate([x, jnp.zeros((_NP - _N, _F), f32)], axis=0)
    hs = jnp.stack([xpad[:, :_HHALF], xpad[:, _HHALF:]], axis=0)
    bpad = jnp.concatenate([batch, jnp.full((_NP - _N,), _NG, jnp.int32)])
    bcol = bpad.reshape(_NP, 1)
    brow = bpad.reshape(1, _NP)

    for p in params["layers"]:
        e = _edge_lin(eap, p["We"][:, :_HHALF], p["We"][:, _HHALF:],
                      p["be"][:_HHALF].reshape(1, _HHALF),
                      p["be"][_HHALF:].reshape(1, _HHALF))
        aggr = _sc_edge(hs, e, idxp)
        scale = jnp.reshape(1.0 + p["eps"], (1, 1)).astype(f32)
        hs = _node_mlp(hs, aggr, scale,
                       p["W1"], p["b1"].reshape(1, _H),
                       p["W2"], p["b2"].reshape(1, _H),
                       p["ln_g"].reshape(1, _H), p["ln_b"].reshape(1, _H))

    return _readout(hs, bcol, brow, global_features,
                    params["gate"], params["cls"])
